# baseline (speedup 1.0000x reference)
import functools

import jax
import jax.numpy as jnp
from jax import lax
from jax.experimental import pallas as pl
from jax.experimental.pallas import tpu as pltpu
from jax.experimental.pallas import tpu_sc as plsc

B = 16384
D = 128
NC = 2
NS = 16
NW = NC * NS
BPW = B // NW
CH = 128
NCH = BPW // CH


def _body(uid_ref, iid_ref, users_ref, items_ref, out_u_ref, out_i_ref,
          idx_u, idx_i, rows_a, rows_b, stage_a, stage_b, sem_g, sem_w):
    cid = lax.axis_index("c")
    sid = lax.axis_index("s")
    wid = sid * NC + cid
    base = wid * BPW

    iota16 = lax.iota(jnp.int32, 16)
    rows_bufs = (rows_a, rows_b)
    stages = (stage_a, stage_b)

    pltpu.sync_copy(uid_ref.at[pl.ds(base, BPW)], idx_u)
    pltpu.sync_copy(iid_ref.at[pl.ds(base, BPW)], idx_i)

    for idx_v, tbl_hbm, out_hbm in (
        (idx_u, users_ref, out_u_ref),
        (idx_i, items_ref, out_i_ref),
    ):
        gathers = [pltpu.async_copy(tbl_hbm.at[idx_v.at[pl.ds(0, CH)]],
                                    rows_bufs[0], sem_g)]
        writebacks = []
        for c in range(NCH):
            gathers[c].wait()
            buf = rows_bufs[c % 2]
            if c + 1 < NCH:
                gathers.append(pltpu.async_copy(
                    tbl_hbm.at[idx_v.at[pl.ds((c + 1) * CH, CH)]],
                    rows_bufs[(c + 1) % 2], sem_g))
            if c >= 2:
                writebacks[c - 2].wait()
            stage = stages[c % 2]
            for j in range(CH // 16):
                stage[0, pl.ds(16 * j, 16)] = (
                    idx_v[pl.ds(c * CH + 16 * j, 16)].astype(jnp.float32))

            @plsc.parallel_loop(0, CH, step=1, unroll=4)
            def _assemble(i):
                col16 = jnp.zeros((16,), jnp.int32) + i
                for j in range(D // 16):
                    plsc.store_scatter(stage,
                                       [iota16 + (1 + 16 * j), col16],
                                       buf[i, pl.ds(16 * j, 16)])

            writebacks.append(pltpu.async_copy(
                stage, out_hbm.at[:, pl.ds(base + c * CH, CH)], sem_w))
        for wb in writebacks[-2:]:
            wb.wait()


@functools.partial(jax.jit, static_argnames=())
def kernel(user_id, item_id, users, items):
    mesh = plsc.VectorSubcoreMesh(core_axis_name="c", subcore_axis_name="s")
    f = pl.kernel(
        _body,
        out_type=(
            jax.ShapeDtypeStruct((D + 1, B), jnp.float32),
            jax.ShapeDtypeStruct((D + 1, B), jnp.float32),
        ),
        mesh=mesh,
        scratch_types=[
            pltpu.VMEM((BPW,), jnp.int32),
            pltpu.VMEM((BPW,), jnp.int32),
            pltpu.VMEM((CH, D), jnp.float32),
            pltpu.VMEM((CH, D), jnp.float32),
            pltpu.VMEM((D + 1, CH), jnp.float32),
            pltpu.VMEM((D + 1, CH), jnp.float32),
            pltpu.SemaphoreType.DMA,
            pltpu.SemaphoreType.DMA,
        ],
        compiler_params=pltpu.CompilerParams(needs_layout_passes=False,
                                             disable_bounds_checks=True),
    )
    out_u_t, out_i_t = f(user_id, item_id, users, items)
    return out_u_t.T, out_i_t.T


# speedup vs baseline: 1.8473x; 1.8473x over previous
import functools

import jax
import jax.numpy as jnp
from jax import lax
from jax.experimental import pallas as pl
from jax.experimental.pallas import tpu as pltpu
from jax.experimental.pallas import tpu_sc as plsc

B = 16384
D = 128
NC = 2
NS = 16
NW = NC * NS
BPW = B // NW
CH = 128
NCH = BPW // CH


def _body(uid_ref, iid_ref, users_ref, items_ref, out_u_ref, out_i_ref,
          idx_u, idx_i, rows_a, rows_b, stage_a, stage_b, sem_g, sem_w):
    cid = lax.axis_index("c")
    sid = lax.axis_index("s")
    wid = sid * NC + cid
    base = wid * BPW

    iota16 = lax.iota(jnp.int32, 16)
    rot = [(iota16 + t) % 16 for t in range(16)]
    rows_bufs = (rows_a, rows_b)
    stages = (stage_a, stage_b)

    pltpu.sync_copy(uid_ref.at[pl.ds(base, BPW)], idx_u)
    pltpu.sync_copy(iid_ref.at[pl.ds(base, BPW)], idx_i)

    for idx_v, tbl_hbm, out_hbm in (
        (idx_u, users_ref, out_u_ref),
        (idx_i, items_ref, out_i_ref),
    ):
        gathers = [pltpu.async_copy(tbl_hbm.at[idx_v.at[pl.ds(0, CH)]],
                                    rows_bufs[0], sem_g)]
        writebacks = []
        for c in range(NCH):
            gathers[c].wait()
            buf = rows_bufs[c % 2]
            if c + 1 < NCH:
                gathers.append(pltpu.async_copy(
                    tbl_hbm.at[idx_v.at[pl.ds((c + 1) * CH, CH)]],
                    rows_bufs[(c + 1) % 2], sem_g))
            if c >= 2:
                writebacks[c - 2].wait()
            stage = stages[c % 2]
            for j in range(CH // 16):
                stage[0, pl.ds(16 * j, 16)] = (
                    idx_v[pl.ds(c * CH + 16 * j, 16)].astype(jnp.float32))

            @plsc.parallel_loop(0, (CH // 16) * (D // 16), step=1, unroll=2)
            def _assemble(blk):
                rows_i = iota16 + (blk // (D // 16)) * 16
                d0 = (blk % (D // 16)) * 16
                for t in range(16):
                    diag = rot[t] + d0
                    vals = plsc.load_gather(buf, [rows_i, diag])
                    plsc.store_scatter(stage, [diag + 1, rows_i], vals)

            writebacks.append(pltpu.async_copy(
                stage, out_hbm.at[:, pl.ds(base + c * CH, CH)], sem_w))
        for wb in writebacks[-2:]:
            wb.wait()


@functools.partial(jax.jit, static_argnames=())
def kernel(user_id, item_id, users, items):
    mesh = plsc.VectorSubcoreMesh(core_axis_name="c", subcore_axis_name="s")
    f = pl.kernel(
        _body,
        out_type=(
            jax.ShapeDtypeStruct((D + 1, B), jnp.float32),
            jax.ShapeDtypeStruct((D + 1, B), jnp.float32),
        ),
        mesh=mesh,
        scratch_types=[
            pltpu.VMEM((BPW,), jnp.int32),
            pltpu.VMEM((BPW,), jnp.int32),
            pltpu.VMEM((CH, D), jnp.float32),
            pltpu.VMEM((CH, D), jnp.float32),
            pltpu.VMEM((D + 1, CH), jnp.float32),
            pltpu.VMEM((D + 1, CH), jnp.float32),
            pltpu.SemaphoreType.DMA,
            pltpu.SemaphoreType.DMA,
        ],
        compiler_params=pltpu.CompilerParams(needs_layout_passes=False,
                                             disable_bounds_checks=True),
    )
    out_u_t, out_i_t = f(user_id, item_id, users, items)
    return out_u_t.T, out_i_t.T


# speedup vs baseline: 1.8929x; 1.0247x over previous
import functools

import jax
import jax.numpy as jnp
from jax import lax
from jax.experimental import pallas as pl
from jax.experimental.pallas import tpu as pltpu
from jax.experimental.pallas import tpu_sc as plsc

B = 16384
D = 128
NC = 2
NS = 16
NW = NC * NS
BPW = B // NW
CH = 128
NCH = BPW // CH


def _body(uid_ref, iid_ref, users_ref, items_ref, out_u_ref, out_i_ref,
          idx_u, idx_i, rows_a, rows_b, rows_c, rows_d,
          stage_a, stage_b, sem_g, sem_w):
    cid = lax.axis_index("c")
    sid = lax.axis_index("s")
    wid = sid * NC + cid
    base = wid * BPW

    iota16 = lax.iota(jnp.int32, 16)
    rot = [(iota16 + t) % 16 for t in range(16)]
    rows_bufs = (rows_a, rows_b, rows_c, rows_d)
    stages = (stage_a, stage_b)

    pltpu.sync_copy(uid_ref.at[pl.ds(base, BPW)], idx_u)
    pltpu.sync_copy(iid_ref.at[pl.ds(base, BPW)], idx_i)

    for idx_v, tbl_hbm, out_hbm in (
        (idx_u, users_ref, out_u_ref),
        (idx_i, items_ref, out_i_ref),
    ):
        gathers = [pltpu.async_copy(tbl_hbm.at[idx_v.at[pl.ds(c * CH, CH)]],
                                    rows_bufs[c], sem_g)
                   for c in range(NCH)]
        writebacks = []
        for c in range(NCH):
            gathers[c].wait()
            buf = rows_bufs[c]
            if c >= 2:
                writebacks[c - 2].wait()
            stage = stages[c % 2]
            for j in range(CH // 16):
                stage[0, pl.ds(16 * j, 16)] = (
                    idx_v[pl.ds(c * CH + 16 * j, 16)].astype(jnp.float32))

            @plsc.parallel_loop(0, (CH // 16) * (D // 16), step=1, unroll=4)
            def _assemble(blk):
                rows_i = iota16 + (blk // (D // 16)) * 16
                d0 = (blk % (D // 16)) * 16
                for t in range(16):
                    diag = rot[t] + d0
                    vals = plsc.load_gather(buf, [rows_i, diag])
                    plsc.store_scatter(stage, [diag + 1, rows_i], vals)

            writebacks.append(pltpu.async_copy(
                stage, out_hbm.at[:, pl.ds(base + c * CH, CH)], sem_w))
        for wb in writebacks[-2:]:
            wb.wait()


@functools.partial(jax.jit, static_argnames=())
def kernel(user_id, item_id, users, items):
    mesh = plsc.VectorSubcoreMesh(core_axis_name="c", subcore_axis_name="s")
    f = pl.kernel(
        _body,
        out_type=(
            jax.ShapeDtypeStruct((D + 1, B), jnp.float32),
            jax.ShapeDtypeStruct((D + 1, B), jnp.float32),
        ),
        mesh=mesh,
        scratch_types=[
            pltpu.VMEM((BPW,), jnp.int32),
            pltpu.VMEM((BPW,), jnp.int32),
            pltpu.VMEM((CH, D), jnp.float32),
            pltpu.VMEM((CH, D), jnp.float32),
            pltpu.VMEM((CH, D), jnp.float32),
            pltpu.VMEM((CH, D), jnp.float32),
            pltpu.VMEM((D + 1, CH), jnp.float32),
            pltpu.VMEM((D + 1, CH), jnp.float32),
            pltpu.SemaphoreType.DMA,
            pltpu.SemaphoreType.DMA,
        ],
        compiler_params=pltpu.CompilerParams(needs_layout_passes=False,
                                             disable_bounds_checks=True),
    )
    out_u_t, out_i_t = f(user_id, item_id, users, items)
    return out_u_t.T, out_i_t.T


# speedup vs baseline: 2.0086x; 1.0611x over previous
import functools

import jax
import jax.numpy as jnp
from jax import lax
from jax.experimental import pallas as pl
from jax.experimental.pallas import tpu as pltpu
from jax.experimental.pallas import tpu_sc as plsc

B = 16384
D = 128
NC = 2
NS = 16
NW = NC * NS
BPW = B // NW
CH = 128
NCH = BPW // CH


def _body(uid_ref, iid_ref, users_ref, items_ref, out_u_ref, out_i_ref,
          idx_u, idx_i, rows_a, rows_b, rows_c, rows_d,
          stage_a, stage_b, sem_g, sem_w):
    cid = lax.axis_index("c")
    sid = lax.axis_index("s")
    wid = sid * NC + cid
    base = wid * BPW

    iota16 = lax.iota(jnp.int32, 16)
    rot = [(iota16 + t) % 16 for t in range(16)]
    rot1 = [r + 1 for r in rot]
    rows_bufs = (rows_a, rows_b, rows_c, rows_d)
    stages = (stage_a, stage_b)

    pltpu.sync_copy(uid_ref.at[pl.ds(base, BPW)], idx_u)
    pltpu.sync_copy(iid_ref.at[pl.ds(base, BPW)], idx_i)

    work = [(idx_u, users_ref, out_u_ref), (idx_i, items_ref, out_i_ref)]

    def chunk_args(k):
        idx_v, tbl_hbm, out_hbm = work[k // NCH]
        c = k % NCH
        return idx_v, tbl_hbm, out_hbm, c

    def fire(k):
        idx_v, tbl_hbm, _, c = chunk_args(k)
        return pltpu.async_copy(tbl_hbm.at[idx_v.at[pl.ds(c * CH, CH)]],
                                rows_bufs[k % 4], sem_g)

    NK = 2 * NCH
    gathers = {k: fire(k) for k in range(3)}
    writebacks = []
    for k in range(NK):
        idx_v, tbl_hbm, out_hbm, c = chunk_args(k)
        gathers[k].wait()
        if k + 3 < NK:
            gathers[k + 3] = fire(k + 3)
        buf = rows_bufs[k % 4]
        if k >= 2:
            writebacks[k - 2].wait()
        stage = stages[k % 2]
        for j in range(CH // 16):
            stage[0, pl.ds(16 * j, 16)] = (
                idx_v[pl.ds(c * CH + 16 * j, 16)].astype(jnp.float32))

        @plsc.parallel_loop(0, (CH // 16) * (D // 16), step=1, unroll=4)
        def _assemble(blk):
            rows_i = iota16 + (blk // (D // 16)) * 16
            d0 = (blk % (D // 16)) * 16
            for t in range(16):
                vals = plsc.load_gather(buf, [rows_i, rot[t] + d0])
                plsc.store_scatter(stage, [rot1[t] + d0, rows_i], vals)

        writebacks.append(pltpu.async_copy(
            stage, out_hbm.at[:, pl.ds(base + c * CH, CH)], sem_w))
    for wb in writebacks[-2:]:
        wb.wait()


@functools.partial(jax.jit, static_argnames=())
def kernel(user_id, item_id, users, items):
    mesh = plsc.VectorSubcoreMesh(core_axis_name="c", subcore_axis_name="s")
    f = pl.kernel(
        _body,
        out_type=(
            jax.ShapeDtypeStruct((D + 1, B), jnp.float32),
            jax.ShapeDtypeStruct((D + 1, B), jnp.float32),
        ),
        mesh=mesh,
        scratch_types=[
            pltpu.VMEM((BPW,), jnp.int32),
            pltpu.VMEM((BPW,), jnp.int32),
            pltpu.VMEM((CH, D), jnp.float32),
            pltpu.VMEM((CH, D), jnp.float32),
            pltpu.VMEM((CH, D), jnp.float32),
            pltpu.VMEM((CH, D), jnp.float32),
            pltpu.VMEM((D + 1, CH), jnp.float32),
            pltpu.VMEM((D + 1, CH), jnp.float32),
            pltpu.SemaphoreType.DMA,
            pltpu.SemaphoreType.DMA,
        ],
        compiler_params=pltpu.CompilerParams(needs_layout_passes=False,
                                             disable_bounds_checks=True),
    )
    out_u_t, out_i_t = f(user_id, item_id, users, items)
    return out_u_t.T, out_i_t.T
